# baseline (speedup 1.0000x reference)
import functools

import jax
import jax.numpy as jnp
from jax import lax
from jax.experimental import pallas as pl
from jax.experimental.pallas import tpu as pltpu
from jax.experimental.pallas import tpu_sc as plsc

_KERNEL_NUM = 64
_SIZE = 1048576
_LANES = 16
_TW = 4096
_NTILES = _SIZE // _TW


def _make_sc_kernel():
    info = plsc.get_sparse_core_info()
    num_workers = info.num_cores * info.num_subcores
    tpw = _NTILES // num_workers

    mesh = plsc.VectorSubcoreMesh(core_axis_name="c", subcore_axis_name="s")

    @functools.partial(
        pl.kernel,
        out_type=jax.ShapeDtypeStruct((_NTILES, _TW), jnp.float32),
        mesh=mesh,
        scratch_types=[
            pltpu.VMEM((_LANES,), jnp.int32),
            pltpu.VMEM((_LANES,), jnp.float32),
            pltpu.VMEM((2 * tpw, _TW), jnp.float32),
            pltpu.VMEM((tpw, _TW), jnp.float32),
            pltpu.SemaphoreType.DMA,
        ],
    )
    def blend(table_hbm, idx_hbm, dist_hbm, out_hbm, idx_v, dist_v,
              rows_v, obuf, sem):
        wid = lax.axis_index("s") * info.num_cores + lax.axis_index("c")

        pltpu.sync_copy(idx_hbm.at[wid], idx_v)
        pltpu.sync_copy(dist_hbm, dist_v)

        dist = dist_v[...]
        one_minus = jnp.float32(1.0) - dist

        del dist, one_minus
        obuf[0, pl.ds(0, _LANES)] = rows_v[0, pl.ds(0, _LANES)]

        pltpu.sync_copy(obuf, out_hbm.at[pl.ds(wid * tpw, tpw)])

    return blend, num_workers, tpw


_blend_sc, _NW, _TPW = _make_sc_kernel()


def kernel(lam, const, pivots):
    del pivots
    scaled = lam[0] * jnp.float32(0.99999) * jnp.float32(_KERNEL_NUM - 1)
    left = jnp.clip(scaled.astype(jnp.int32), 0, _KERNEL_NUM - 2)
    dist = (left + 1).astype(jnp.float32) - scaled
    dist16 = jnp.broadcast_to(dist, (_LANES,))

    k = jnp.arange(_LANES, dtype=jnp.int32)
    off = (k % _TPW) + (k // _TPW) * _NTILES
    idx = (left * _NTILES + jnp.arange(_NW, dtype=jnp.int32)[:, None] * _TPW
           + off[None, :])

    table = const.reshape(_KERNEL_NUM * _NTILES, _TW)
    out = _blend_sc(table, idx, dist16)
    return out.reshape(1, _SIZE)


# speedup vs baseline: 2.3160x; 2.3160x over previous
import jax
import jax.numpy as jnp
from jax.experimental import pallas as pl
from jax.experimental.pallas import tpu as pltpu

_KERNEL_NUM = 64
_SIZE = 1048576
_SUB = 8
_W = _SIZE // _SUB
_BT = 16384


def _body(lidx_ref, lam_ref, lrow_ref, rrow_ref, out_ref):
    del lidx_ref
    scaled = lam_ref[0] * jnp.float32(0.99999) * jnp.float32(_KERNEL_NUM - 1)
    lf = jnp.clip(scaled.astype(jnp.int32), 0, _KERNEL_NUM - 2)
    dist = (lf + 1).astype(jnp.float32) - scaled
    out_ref[...] = (lrow_ref[...] * dist
                    + rrow_ref[...] * (jnp.float32(1.0) - dist))


_grid_spec = pltpu.PrefetchScalarGridSpec(
    num_scalar_prefetch=1,
    grid=(_W // _BT,),
    in_specs=[
        pl.BlockSpec(memory_space=pltpu.SMEM),
        pl.BlockSpec((1, _SUB, _BT), lambda j, lidx: (lidx[0], 0, j)),
        pl.BlockSpec((1, _SUB, _BT), lambda j, lidx: (lidx[0] + 1, 0, j)),
    ],
    out_specs=pl.BlockSpec((1, _SUB, _BT), lambda j, lidx: (0, 0, j)),
)

_blend = pl.pallas_call(
    _body,
    grid_spec=_grid_spec,
    out_shape=jax.ShapeDtypeStruct((1, _SUB, _W), jnp.float32),
)


def kernel(lam, const, pivots):
    del pivots
    scaled = lam * jnp.float32(0.99999) * jnp.float32(_KERNEL_NUM - 1)
    lidx = jnp.clip(scaled.astype(jnp.int32), 0, _KERNEL_NUM - 2)
    constv = const.reshape(_KERNEL_NUM, _SUB, _W)
    out = _blend(lidx, lam, constv, constv)
    return out.reshape(1, _SIZE)


# speedup vs baseline: 59.6421x; 25.7519x over previous
import jax
import jax.numpy as jnp
from jax.experimental import pallas as pl
from jax.experimental.pallas import tpu as pltpu

_KERNEL_NUM = 64
_SIZE = 1048576
_BT = 131072
_GRID = _SIZE // _BT


def _body(lidx_ref, lam_ref, const_ref, out_ref, ibuf, sems):
    j = pl.program_id(0)
    left = lidx_ref[0]

    def start_in(jj, slot):
        col = pl.ds(jj * _BT, _BT)
        pltpu.make_async_copy(
            const_ref.at[pl.ds(left, 1), col],
            ibuf.at[slot, 0], sems.at[slot, 0]).start()
        pltpu.make_async_copy(
            const_ref.at[pl.ds(left + 1, 1), col],
            ibuf.at[slot, 1], sems.at[slot, 1]).start()

    @pl.when(j == 0)
    def _():
        start_in(0, 0)

    @pl.when(j + 1 < _GRID)
    def _():
        start_in(j + 1, (j + 1) % 2)

    slot = j % 2
    pltpu.make_async_copy(
        const_ref.at[pl.ds(left, 1), pl.ds(0, _BT)],
        ibuf.at[slot, 0], sems.at[slot, 0]).wait()
    pltpu.make_async_copy(
        const_ref.at[pl.ds(left + 1, 1), pl.ds(0, _BT)],
        ibuf.at[slot, 1], sems.at[slot, 1]).wait()

    scaled = lam_ref[0] * jnp.float32(0.99999) * jnp.float32(_KERNEL_NUM - 1)
    lf = jnp.clip(scaled.astype(jnp.int32), 0, _KERNEL_NUM - 2)
    dist = (lf + 1).astype(jnp.float32) - scaled
    out_ref[...] = (ibuf[slot, 0] * dist
                    + ibuf[slot, 1] * (jnp.float32(1.0) - dist))


_grid_spec = pltpu.PrefetchScalarGridSpec(
    num_scalar_prefetch=1,
    grid=(_GRID,),
    in_specs=[
        pl.BlockSpec(memory_space=pltpu.SMEM),
        pl.BlockSpec(memory_space=pl.MemorySpace.ANY),
    ],
    out_specs=pl.BlockSpec((1, _BT), lambda j, lidx: (0, j)),
    scratch_shapes=[
        pltpu.VMEM((2, 2, 1, _BT), jnp.float32),
        pltpu.SemaphoreType.DMA((2, 2)),
    ],
)

_blend = pl.pallas_call(
    _body,
    grid_spec=_grid_spec,
    out_shape=jax.ShapeDtypeStruct((1, _SIZE), jnp.float32),
)


def kernel(lam, const, pivots):
    del pivots
    scaled = lam * jnp.float32(0.99999) * jnp.float32(_KERNEL_NUM - 1)
    lidx = jnp.clip(scaled.astype(jnp.int32), 0, _KERNEL_NUM - 2)
    return _blend(lidx, lam, const)


# speedup vs baseline: 76.6037x; 1.2844x over previous
import jax
import jax.numpy as jnp
from jax.experimental import pallas as pl
from jax.experimental.pallas import tpu as pltpu

_KERNEL_NUM = 64
_SIZE = 1048576
_BT = 262144
_GRID = _SIZE // _BT


def _body(lidx_ref, lam_ref, const_ref, out_ref, ibuf, sems):
    j = pl.program_id(0)
    left = lidx_ref[0]

    def start_in(jj, slot):
        col = pl.ds(jj * _BT, _BT)
        pltpu.make_async_copy(
            const_ref.at[pl.ds(left, 1), col],
            ibuf.at[slot, 0], sems.at[slot, 0]).start()
        pltpu.make_async_copy(
            const_ref.at[pl.ds(left + 1, 1), col],
            ibuf.at[slot, 1], sems.at[slot, 1]).start()

    @pl.when(j == 0)
    def _():
        start_in(0, 0)

    @pl.when(j + 1 < _GRID)
    def _():
        start_in(j + 1, (j + 1) % 2)

    slot = j % 2
    pltpu.make_async_copy(
        const_ref.at[pl.ds(left, 1), pl.ds(0, _BT)],
        ibuf.at[slot, 0], sems.at[slot, 0]).wait()
    pltpu.make_async_copy(
        const_ref.at[pl.ds(left + 1, 1), pl.ds(0, _BT)],
        ibuf.at[slot, 1], sems.at[slot, 1]).wait()

    scaled = lam_ref[0] * jnp.float32(0.99999) * jnp.float32(_KERNEL_NUM - 1)
    lf = jnp.clip(scaled.astype(jnp.int32), 0, _KERNEL_NUM - 2)
    dist = (lf + 1).astype(jnp.float32) - scaled
    out_ref[...] = (ibuf[slot, 0] * dist
                    + ibuf[slot, 1] * (jnp.float32(1.0) - dist))


_grid_spec = pltpu.PrefetchScalarGridSpec(
    num_scalar_prefetch=1,
    grid=(_GRID,),
    in_specs=[
        pl.BlockSpec(memory_space=pltpu.SMEM),
        pl.BlockSpec(memory_space=pl.MemorySpace.ANY),
    ],
    out_specs=pl.BlockSpec((1, _BT), lambda j, lidx: (0, j)),
    scratch_shapes=[
        pltpu.VMEM((2, 2, 1, _BT), jnp.float32),
        pltpu.SemaphoreType.DMA((2, 2)),
    ],
)

_blend = pl.pallas_call(
    _body,
    grid_spec=_grid_spec,
    out_shape=jax.ShapeDtypeStruct((1, _SIZE), jnp.float32),
)


def kernel(lam, const, pivots):
    del pivots
    scaled = lam * jnp.float32(0.99999) * jnp.float32(_KERNEL_NUM - 1)
    lidx = jnp.clip(scaled.astype(jnp.int32), 0, _KERNEL_NUM - 2)
    return _blend(lidx, lam, const)


# speedup vs baseline: 91.3394x; 1.1924x over previous
import jax
import jax.numpy as jnp
from jax.experimental import pallas as pl
from jax.experimental.pallas import tpu as pltpu

_KERNEL_NUM = 64
_SIZE = 1048576
_BT = 524288
_GRID = _SIZE // _BT


def _body(lidx_ref, lam_ref, const_ref, out_ref, ibuf, sems):
    j = pl.program_id(0)
    left = lidx_ref[0]

    def start_in(jj, slot):
        col = pl.ds(jj * _BT, _BT)
        pltpu.make_async_copy(
            const_ref.at[pl.ds(left, 1), col],
            ibuf.at[slot, 0], sems.at[slot, 0]).start()
        pltpu.make_async_copy(
            const_ref.at[pl.ds(left + 1, 1), col],
            ibuf.at[slot, 1], sems.at[slot, 1]).start()

    @pl.when(j == 0)
    def _():
        start_in(0, 0)

    @pl.when(j + 1 < _GRID)
    def _():
        start_in(j + 1, (j + 1) % 2)

    slot = j % 2
    pltpu.make_async_copy(
        const_ref.at[pl.ds(left, 1), pl.ds(0, _BT)],
        ibuf.at[slot, 0], sems.at[slot, 0]).wait()
    pltpu.make_async_copy(
        const_ref.at[pl.ds(left + 1, 1), pl.ds(0, _BT)],
        ibuf.at[slot, 1], sems.at[slot, 1]).wait()

    scaled = lam_ref[0] * jnp.float32(0.99999) * jnp.float32(_KERNEL_NUM - 1)
    lf = jnp.clip(scaled.astype(jnp.int32), 0, _KERNEL_NUM - 2)
    dist = (lf + 1).astype(jnp.float32) - scaled
    out_ref[...] = (ibuf[slot, 0] * dist
                    + ibuf[slot, 1] * (jnp.float32(1.0) - dist))


_grid_spec = pltpu.PrefetchScalarGridSpec(
    num_scalar_prefetch=1,
    grid=(_GRID,),
    in_specs=[
        pl.BlockSpec(memory_space=pltpu.SMEM),
        pl.BlockSpec(memory_space=pl.MemorySpace.ANY),
    ],
    out_specs=pl.BlockSpec((1, _BT), lambda j, lidx: (0, j)),
    scratch_shapes=[
        pltpu.VMEM((2, 2, 1, _BT), jnp.float32),
        pltpu.SemaphoreType.DMA((2, 2)),
    ],
)

_blend = pl.pallas_call(
    _body,
    grid_spec=_grid_spec,
    out_shape=jax.ShapeDtypeStruct((1, _SIZE), jnp.float32),
)


def kernel(lam, const, pivots):
    del pivots
    scaled = lam * jnp.float32(0.99999) * jnp.float32(_KERNEL_NUM - 1)
    lidx = jnp.clip(scaled.astype(jnp.int32), 0, _KERNEL_NUM - 2)
    return _blend(lidx, lam, const)
